# Initial kernel scaffold; baseline (speedup 1.0000x reference)
import jax
import jax.numpy as jnp
from jax.experimental import pallas as pl


def kernel(x, pos, W_first, b_first, W_mid, b_mid, W_last, b_last):
    raise NotImplementedError("write your pallas kernel here")


# speedup vs baseline: 5.3321x; 5.3321x over previous
import functools

import jax
import jax.numpy as jnp
from jax import lax
from jax.experimental import pallas as pl
from jax.experimental.pallas import tpu as pltpu
from jax.experimental.pallas import tpu_sc as plsc

_K = 16
_TILE = 256


_CW = 512


def _knn_kernel(n_pts, posTc_ref, pos_ref, x_ref, w1nT_ref, idx_ref, c_ref,
                d2_ref):
    b = pl.program_id(0)
    t = pl.program_id(1)
    nch = n_pts // _CW
    pos_t = pos_ref[0]
    sq_t = jnp.sum(pos_t * pos_t, axis=1, keepdims=True)
    inf = jnp.float32(jnp.inf)
    row_g = t * _TILE + lax.broadcasted_iota(jnp.int32, (_TILE, _CW), 0)

    def build(ch, _):
        posc = posTc_ref[0, :, ch]
        sq_c = jnp.sum(posc * posc, axis=0, keepdims=True)
        dot = lax.dot_general(pos_t, posc, (((1,), (0,)), ((), ())),
                              precision=lax.Precision.HIGHEST,
                              preferred_element_type=jnp.float32)
        d2c = (sq_t + sq_c) - 2.0 * dot
        col = ch * _CW + lax.broadcasted_iota(jnp.int32, (_TILE, _CW), 1)
        d2_ref[ch] = jnp.where(col == row_g, inf, d2c)
        return 0

    lax.fori_loop(0, nch, build, 0, unroll=False)

    big = jnp.int32(n_pts)
    m_p = jnp.full((_TILE, 1), -inf, jnp.float32)
    c_p = jnp.full((_TILE, 1), -1, jnp.int32)
    for k in range(_K):
        def scan(ch, carry):
            m_run, c_run = carry
            v = d2_ref[ch]
            col = ch * _CW + lax.broadcasted_iota(jnp.int32, (_TILE, _CW), 1)
            ok = (v > m_p) | ((v == m_p) & (col > c_p))
            vm = jnp.where(ok, v, inf)
            cm = jnp.min(vm, axis=1, keepdims=True)
            cc = jnp.min(jnp.where(vm == cm, col, big), axis=1, keepdims=True)
            better = (cm < m_run) | ((cm == m_run) & (cc < c_run))
            return (jnp.where(better, cm, m_run),
                    jnp.where(better, cc, c_run))

        m_p, c_p = lax.fori_loop(
            0, nch, scan,
            (jnp.full((_TILE, 1), inf, jnp.float32),
             jnp.full((_TILE, 1), big, jnp.int32)), unroll=False)
        idx_ref[0, :, k:k + 1] = c_p + b * n_pts

    c_t = lax.dot_general(x_ref[0], w1nT_ref[...],
                          (((1,), (0,)), ((), ())),
                          precision=lax.Precision.HIGHEST,
                          preferred_element_type=jnp.float32)
    c_ref[0] = jnp.concatenate(
        [c_t, jnp.zeros((c_t.shape[0], 128 - c_t.shape[1]), jnp.float32)],
        axis=1)


def _mlp_kernel(x_ref, g_ref, w1xT_ref, b1_ref, w2hT_ref, w2xT_ref, b2_ref,
                w3aT_ref, w3bT_ref, w3xT_ref, b3_ref, out_ref):
    def mm(u, wT_ref):
        return lax.dot_general(u, wT_ref[...], (((1,), (0,)), ((), ())),
                               precision=lax.Precision.HIGHEST,
                               preferred_element_type=jnp.float32)
    x_t = x_ref[0]
    a = mm(x_t, w1xT_ref) + b1_ref[...]
    d = mm(x_t, w2xT_ref) + b2_ref[...]
    e = mm(x_t, w3xT_ref) + b3_ref[...]
    ninf = jnp.float32(-jnp.inf)
    m1 = jnp.full(a.shape, ninf, jnp.float32)
    m2 = jnp.full(a.shape, ninf, jnp.float32)
    m3 = jnp.full(a.shape, ninf, jnp.float32)
    gr = a.shape[1]
    for k in range(_K):
        g_k = g_ref[0, k][:, :gr]
        h1 = jnp.maximum(a + g_k, 0.0)
        h2 = jnp.maximum(mm(h1, w2hT_ref) + d, 0.0)
        h3 = mm(h2, w3aT_ref) + mm(h1, w3bT_ref) + e
        m1 = jnp.maximum(m1, h1)
        m2 = jnp.maximum(m2, h2)
        m3 = jnp.maximum(m3, h3)
    out_ref[0] = jnp.concatenate([m3, m2, m1, x_t], axis=1)


def _make_sc_gather(n_rows, gr, total):
    info = plsc.get_sparse_core_info()
    nw = info.num_cores * info.num_subcores
    per_w = total // nw
    chunk = min(per_w, 512)
    n_ch = per_w // chunk

    mesh = plsc.VectorSubcoreMesh(core_axis_name="c", subcore_axis_name="s")

    @functools.partial(
        pl.kernel, mesh=mesh,
        out_type=jax.ShapeDtypeStruct((total, 128), jnp.float32),
        scratch_types=[
            pltpu.VMEM((chunk,), jnp.int32),
            pltpu.VMEM((chunk, 128), jnp.float32),
            pltpu.SemaphoreType.DMA,
        ],
    )
    def gather(table_hbm, idx_hbm, out_hbm, idx_v, rows_v, sem):
        wid = lax.axis_index("s") * info.num_cores + lax.axis_index("c")
        base = wid * per_w
        for ci in range(n_ch):
            off = base + ci * chunk
            pltpu.sync_copy(idx_hbm.at[pl.ds(off, chunk)], idx_v)
            pltpu.async_copy(table_hbm.at[idx_v], rows_v, sem).wait()
            pltpu.sync_copy(rows_v, out_hbm.at[pl.ds(off, chunk)])

    return gather


def kernel(x, pos, W_first, b_first, W_mid, b_mid, W_last, b_last):
    B, N, D = x.shape
    GR = W_first.shape[0]
    NT = N // _TILE

    W1a, W1b, W1c = W_first[:, :D], W_first[:, D:2 * D], W_first[:, 2 * D:]
    w1xT = (W1a - W1c).T
    w1nT = (W1b + W1c).T
    w2hT = W_mid[:, :GR].T
    w2xT = W_mid[:, GR:].T
    w3aT = W_last[:, :GR].T
    w3bT = W_last[:, GR:2 * GR].T
    w3xT = W_last[:, 2 * GR:].T
    b1 = b_first.reshape(1, GR)
    b2 = b_mid.reshape(1, GR)
    b3 = b_last.reshape(1, GR)
    NCH = N // _CW
    posTc = jnp.transpose(pos, (0, 2, 1)).reshape(B, 3, NCH, _CW)

    idx_g, c = pl.pallas_call(
        functools.partial(_knn_kernel, N),
        grid=(B, NT),
        in_specs=[
            pl.BlockSpec((1, 3, NCH, _CW), lambda b, t: (b, 0, 0, 0)),
            pl.BlockSpec((1, _TILE, 3), lambda b, t: (b, t, 0)),
            pl.BlockSpec((1, _TILE, D), lambda b, t: (b, t, 0)),
            pl.BlockSpec((D, GR), lambda b, t: (0, 0)),
        ],
        out_specs=[
            pl.BlockSpec((1, _TILE, _K), lambda b, t: (b, t, 0)),
            pl.BlockSpec((1, _TILE, 128), lambda b, t: (b, t, 0)),
        ],
        out_shape=[
            jax.ShapeDtypeStruct((B, N, _K), jnp.int32),
            jax.ShapeDtypeStruct((B, N, 128), jnp.float32),
        ],
        scratch_shapes=[pltpu.VMEM((NCH, _TILE, _CW), jnp.float32)],
    )(posTc, pos, x, w1nT)

    idx_flat = jnp.transpose(idx_g, (0, 2, 1)).reshape(-1)
    total = B * _K * N
    g_flat = _make_sc_gather(B * N, GR, total)(
        c.reshape(B * N, 128), idx_flat)
    g = g_flat.reshape(B, _K, N, 128)

    out = pl.pallas_call(
        _mlp_kernel,
        grid=(B, NT),
        in_specs=[
            pl.BlockSpec((1, _TILE, D), lambda b, t: (b, t, 0)),
            pl.BlockSpec((1, _K, _TILE, 128), lambda b, t: (b, 0, t, 0)),
            pl.BlockSpec((D, GR), lambda b, t: (0, 0)),
            pl.BlockSpec((1, GR), lambda b, t: (0, 0)),
            pl.BlockSpec((GR, GR), lambda b, t: (0, 0)),
            pl.BlockSpec((D, GR), lambda b, t: (0, 0)),
            pl.BlockSpec((1, GR), lambda b, t: (0, 0)),
            pl.BlockSpec((GR, GR), lambda b, t: (0, 0)),
            pl.BlockSpec((GR, GR), lambda b, t: (0, 0)),
            pl.BlockSpec((D, GR), lambda b, t: (0, 0)),
            pl.BlockSpec((1, GR), lambda b, t: (0, 0)),
        ],
        out_specs=pl.BlockSpec((1, _TILE, D + 3 * GR), lambda b, t: (b, t, 0)),
        out_shape=jax.ShapeDtypeStruct((B, N, D + 3 * GR), jnp.float32),
    )(x, g, w1xT, b1, w2hT, w2xT, b2, w3aT, w3bT, w3xT, b3)
    return out


# speedup vs baseline: 6.6734x; 1.2516x over previous
import functools

import jax
import jax.numpy as jnp
from jax import lax
from jax.experimental import pallas as pl
from jax.experimental.pallas import tpu as pltpu
from jax.experimental.pallas import tpu_sc as plsc

_K = 16
_TILE = 256


_CW = 512


def _knn_kernel(n_pts, posTc_ref, pos_ref, x_ref, w1nT_ref, idx_ref, c_ref,
                d2_ref):
    b = pl.program_id(0)
    t = pl.program_id(1)
    nch = n_pts // _CW
    pos_t = pos_ref[0]
    sq_t = jnp.sum(pos_t * pos_t, axis=1, keepdims=True)
    inf = jnp.float32(jnp.inf)
    row_g = t * _TILE + lax.broadcasted_iota(jnp.int32, (_TILE, _CW), 0)

    def build(ch, _):
        posc = posTc_ref[0, :, ch]
        sq_c = jnp.sum(posc * posc, axis=0, keepdims=True)
        dot = lax.dot_general(pos_t, posc, (((1,), (0,)), ((), ())),
                              precision=lax.Precision.HIGHEST,
                              preferred_element_type=jnp.float32)
        d2c = (sq_t + sq_c) - 2.0 * dot
        col = ch * _CW + lax.broadcasted_iota(jnp.int32, (_TILE, _CW), 1)
        d2_ref[ch] = jnp.where(col == row_g, inf, d2c)
        return 0

    lax.fori_loop(0, nch, build, 0, unroll=False)

    big = jnp.int32(n_pts)
    m_p = jnp.full((_TILE, 1), -inf, jnp.float32)
    c_p = jnp.full((_TILE, 1), -1, jnp.int32)
    for k in range(_K):
        def scan(ch, carry):
            vacc, cacc = carry
            v = d2_ref[ch]
            col = ch * _CW + lax.broadcasted_iota(jnp.int32, (_TILE, _CW), 1)
            ok = (v > m_p) | ((v == m_p) & (col > c_p))
            vm = jnp.where(ok, v, inf)
            pick = vm < vacc
            return (jnp.where(pick, vm, vacc), jnp.where(pick, col, cacc))

        vacc, cacc = lax.fori_loop(
            0, nch, scan,
            (jnp.full((_TILE, _CW), inf, jnp.float32),
             jnp.full((_TILE, _CW), big, jnp.int32)), unroll=False)
        m_p = jnp.min(vacc, axis=1, keepdims=True)
        c_p = jnp.min(jnp.where(vacc == m_p, cacc, big), axis=1,
                      keepdims=True)
        idx_ref[0, :, k:k + 1] = c_p + b * n_pts

    c_t = lax.dot_general(x_ref[0], w1nT_ref[...],
                          (((1,), (0,)), ((), ())),
                          precision=lax.Precision.HIGHEST,
                          preferred_element_type=jnp.float32)
    c_ref[0] = jnp.concatenate(
        [c_t, jnp.zeros((c_t.shape[0], 128 - c_t.shape[1]), jnp.float32)],
        axis=1)


def _mlp_kernel(x_ref, g_ref, w1xT_ref, b1_ref, w2hT_ref, w2xT_ref, b2_ref,
                w3aT_ref, w3bT_ref, w3xT_ref, b3_ref, out_ref):
    def mm(u, wT_ref):
        return lax.dot_general(u, wT_ref[...], (((1,), (0,)), ((), ())),
                               precision=lax.Precision.HIGHEST,
                               preferred_element_type=jnp.float32)
    x_t = x_ref[0]
    a = mm(x_t, w1xT_ref) + b1_ref[...]
    d = mm(x_t, w2xT_ref) + b2_ref[...]
    e = mm(x_t, w3xT_ref) + b3_ref[...]
    ninf = jnp.float32(-jnp.inf)
    m1 = jnp.full(a.shape, ninf, jnp.float32)
    m2 = jnp.full(a.shape, ninf, jnp.float32)
    m3 = jnp.full(a.shape, ninf, jnp.float32)
    gr = a.shape[1]
    for k in range(_K):
        g_k = g_ref[0, k][:, :gr]
        h1 = jnp.maximum(a + g_k, 0.0)
        h2 = jnp.maximum(mm(h1, w2hT_ref) + d, 0.0)
        h3 = mm(h2, w3aT_ref) + mm(h1, w3bT_ref) + e
        m1 = jnp.maximum(m1, h1)
        m2 = jnp.maximum(m2, h2)
        m3 = jnp.maximum(m3, h3)
    out_ref[0] = jnp.concatenate([m3, m2, m1, x_t], axis=1)


def _make_sc_gather(n_rows, gr, total):
    info = plsc.get_sparse_core_info()
    nw = info.num_cores * info.num_subcores
    per_w = total // nw
    chunk = min(per_w, 512)
    n_ch = per_w // chunk

    mesh = plsc.VectorSubcoreMesh(core_axis_name="c", subcore_axis_name="s")

    @functools.partial(
        pl.kernel, mesh=mesh,
        out_type=jax.ShapeDtypeStruct((total, 128), jnp.float32),
        scratch_types=[
            pltpu.VMEM((chunk,), jnp.int32),
            pltpu.VMEM((chunk, 128), jnp.float32),
            pltpu.SemaphoreType.DMA,
        ],
    )
    def gather(table_hbm, idx_hbm, out_hbm, idx_v, rows_v, sem):
        wid = lax.axis_index("s") * info.num_cores + lax.axis_index("c")
        base = wid * per_w
        for ci in range(n_ch):
            off = base + ci * chunk
            pltpu.sync_copy(idx_hbm.at[pl.ds(off, chunk)], idx_v)
            pltpu.async_copy(table_hbm.at[idx_v], rows_v, sem).wait()
            pltpu.sync_copy(rows_v, out_hbm.at[pl.ds(off, chunk)])

    return gather


def kernel(x, pos, W_first, b_first, W_mid, b_mid, W_last, b_last):
    B, N, D = x.shape
    GR = W_first.shape[0]
    NT = N // _TILE

    W1a, W1b, W1c = W_first[:, :D], W_first[:, D:2 * D], W_first[:, 2 * D:]
    w1xT = (W1a - W1c).T
    w1nT = (W1b + W1c).T
    w2hT = W_mid[:, :GR].T
    w2xT = W_mid[:, GR:].T
    w3aT = W_last[:, :GR].T
    w3bT = W_last[:, GR:2 * GR].T
    w3xT = W_last[:, 2 * GR:].T
    b1 = b_first.reshape(1, GR)
    b2 = b_mid.reshape(1, GR)
    b3 = b_last.reshape(1, GR)
    NCH = N // _CW
    posTc = jnp.transpose(pos, (0, 2, 1)).reshape(B, 3, NCH, _CW)

    idx_g, c = pl.pallas_call(
        functools.partial(_knn_kernel, N),
        grid=(B, NT),
        in_specs=[
            pl.BlockSpec((1, 3, NCH, _CW), lambda b, t: (b, 0, 0, 0)),
            pl.BlockSpec((1, _TILE, 3), lambda b, t: (b, t, 0)),
            pl.BlockSpec((1, _TILE, D), lambda b, t: (b, t, 0)),
            pl.BlockSpec((D, GR), lambda b, t: (0, 0)),
        ],
        out_specs=[
            pl.BlockSpec((1, _TILE, _K), lambda b, t: (b, t, 0)),
            pl.BlockSpec((1, _TILE, 128), lambda b, t: (b, t, 0)),
        ],
        out_shape=[
            jax.ShapeDtypeStruct((B, N, _K), jnp.int32),
            jax.ShapeDtypeStruct((B, N, 128), jnp.float32),
        ],
        scratch_shapes=[pltpu.VMEM((NCH, _TILE, _CW), jnp.float32)],
    )(posTc, pos, x, w1nT)

    idx_flat = jnp.transpose(idx_g, (0, 2, 1)).reshape(-1)
    total = B * _K * N
    g_flat = _make_sc_gather(B * N, GR, total)(
        c.reshape(B * N, 128), idx_flat)
    g = g_flat.reshape(B, _K, N, 128)

    out = pl.pallas_call(
        _mlp_kernel,
        grid=(B, NT),
        in_specs=[
            pl.BlockSpec((1, _TILE, D), lambda b, t: (b, t, 0)),
            pl.BlockSpec((1, _K, _TILE, 128), lambda b, t: (b, 0, t, 0)),
            pl.BlockSpec((D, GR), lambda b, t: (0, 0)),
            pl.BlockSpec((1, GR), lambda b, t: (0, 0)),
            pl.BlockSpec((GR, GR), lambda b, t: (0, 0)),
            pl.BlockSpec((D, GR), lambda b, t: (0, 0)),
            pl.BlockSpec((1, GR), lambda b, t: (0, 0)),
            pl.BlockSpec((GR, GR), lambda b, t: (0, 0)),
            pl.BlockSpec((GR, GR), lambda b, t: (0, 0)),
            pl.BlockSpec((D, GR), lambda b, t: (0, 0)),
            pl.BlockSpec((1, GR), lambda b, t: (0, 0)),
        ],
        out_specs=pl.BlockSpec((1, _TILE, D + 3 * GR), lambda b, t: (b, t, 0)),
        out_shape=jax.ShapeDtypeStruct((B, N, D + 3 * GR), jnp.float32),
    )(x, g, w1xT, b1, w2hT, w2xT, b2, w3aT, w3bT, w3xT, b3)
    return out
